# baseline (speedup 1.0000x reference)
import functools

import jax
import jax.numpy as jnp
from jax import lax
from jax.experimental import pallas as pl
from jax.experimental.pallas import tpu as pltpu
from jax.experimental.pallas import tpu_sc as plsc

N = 10000
E = 320000
D = 128

NW = 32
NSUB = 16
EPW = E // NW
CH = 128
ECH = 128
NCHUNK = 80
EPW_PAD = NCHUNK * ECH
NCH_D = NCHUNK
N_PAD = 10112
RPT = N_PAD // NSUB

_mesh = plsc.VectorSubcoreMesh(core_axis_name="c", subcore_axis_name="s")


@functools.partial(
    pl.kernel,
    mesh=_mesh,
    out_type=tuple(jax.ShapeDtypeStruct((N_PAD,), jnp.float32)
                   for _ in range(4)),
    scratch_types=[
        pltpu.VMEM((NCH_D, 2, CH), jnp.int32),
        pltpu.VMEM((CH,), jnp.float32),
        pltpu.VMEM((RPT,), jnp.float32),
        pltpu.VMEM_SHARED((N_PAD,), jnp.float32),
        pltpu.VMEM_SHARED((N_PAD,), jnp.float32),
    ],
)
def _deg_kernel(sd_hbm, ones_hbm, zrow_hbm,
                dego0_hbm, degi0_hbm, dego1_hbm, degi1_hbm,
                idx_v, ones_v, vbuf, dego, degi):
    c = lax.axis_index("c")
    s = lax.axis_index("s")
    wid = c * NSUB + s
    base = s * RPT
    pltpu.sync_copy(ones_hbm, ones_v)
    pltpu.sync_copy(zrow_hbm, vbuf)
    pltpu.sync_copy(vbuf, dego.at[pl.ds(base, RPT)])
    pltpu.sync_copy(vbuf, degi.at[pl.ds(base, RPT)])
    plsc.subcore_barrier()

    pltpu.sync_copy(sd_hbm.at[wid], idx_v)

    @pl.loop(0, NCH_D)
    def _(j):
        pltpu.sync_copy(ones_v, dego.at[idx_v.at[j, 0]], add=True)
        pltpu.sync_copy(ones_v, degi.at[idx_v.at[j, 1]], add=True)

    plsc.subcore_barrier()

    @pl.when(c == 0)
    def _():
        pltpu.sync_copy(dego.at[pl.ds(base, RPT)], vbuf)
        pltpu.sync_copy(vbuf, dego0_hbm.at[pl.ds(base, RPT)])
        pltpu.sync_copy(degi.at[pl.ds(base, RPT)], vbuf)
        pltpu.sync_copy(vbuf, degi0_hbm.at[pl.ds(base, RPT)])

    @pl.when(c == 1)
    def _():
        pltpu.sync_copy(dego.at[pl.ds(base, RPT)], vbuf)
        pltpu.sync_copy(vbuf, dego1_hbm.at[pl.ds(base, RPT)])
        pltpu.sync_copy(degi.at[pl.ds(base, RPT)], vbuf)
        pltpu.sync_copy(vbuf, degi1_hbm.at[pl.ds(base, RPT)])


@functools.partial(
    pl.kernel,
    mesh=_mesh,
    out_type=jax.ShapeDtypeStruct((2, N_PAD, D), jnp.float32),
    scratch_types=[
        pltpu.VMEM((NCHUNK, 2, ECH), jnp.int32),
        pltpu.VMEM((ECH, D), jnp.float32),
        pltpu.VMEM_SHARED((N_PAD, D), jnp.float32),
    ],
)
def _agg_kernel(h_hbm, sd_hbm, zrows_hbm, out_hbm, idxb, rows, acc):
    c = lax.axis_index("c")
    s = lax.axis_index("s")
    wid = c * NSUB + s
    base = s * RPT
    pltpu.sync_copy(zrows_hbm, rows)
    for k in range(-(-RPT // ECH)):
        sz = min(ECH, RPT - k * ECH)
        pltpu.sync_copy(rows.at[pl.ds(0, sz)],
                        acc.at[pl.ds(base + k * ECH, sz)])
    plsc.subcore_barrier()

    pltpu.sync_copy(sd_hbm.at[wid], idxb)

    @pl.loop(0, NCHUNK)
    def _(j):
        pltpu.sync_copy(h_hbm.at[idxb.at[j, 0]], rows)
        pltpu.sync_copy(rows, acc.at[idxb.at[j, 1]], add=True)

    plsc.subcore_barrier()
    for k in range(-(-RPT // ECH)):
        sz = min(ECH, RPT - k * ECH)
        pltpu.sync_copy(acc.at[pl.ds(base + k * ECH, sz)],
                        rows.at[pl.ds(0, sz)])
        pltpu.sync_copy(rows.at[pl.ds(0, sz)],
                        out_hbm.at[c, pl.ds(base + k * ECH, sz)])


def _norm_body(do0, di0, do1, di1, no_ref, ni_ref):
    no_ref[...] = lax.rsqrt(jnp.maximum(do0[...] + do1[...], 1.0))
    ni_ref[...] = lax.rsqrt(jnp.maximum(di0[...] + di1[...], 1.0))


_norm = pl.pallas_call(
    _norm_body,
    out_shape=(jax.ShapeDtypeStruct((N_PAD,), jnp.float32),
               jax.ShapeDtypeStruct((N_PAD,), jnp.float32)),
)


def _mm1_body(x_ref, w_ref, no_ref, o_ref):
    h = jnp.dot(x_ref[...], w_ref[...], preferred_element_type=jnp.float32)
    o_ref[...] = h * no_ref[...]


_mm1 = pl.pallas_call(
    _mm1_body,
    out_shape=jax.ShapeDtypeStruct((N_PAD, D), jnp.float32),
)


def _mm2_body(p_ref, ni_ref, b_ref, w_ref, no_ref, o_ref):
    x = (p_ref[0] + p_ref[1]) * ni_ref[...] + b_ref[...]
    h = jnp.dot(x, w_ref[...], preferred_element_type=jnp.float32)
    o_ref[...] = h * no_ref[...]


_mm2 = pl.pallas_call(
    _mm2_body,
    out_shape=jax.ShapeDtypeStruct((N_PAD, D), jnp.float32),
)


def _fin_body(p_ref, ni_ref, b_ref, o_ref):
    o_ref[...] = (p_ref[0] + p_ref[1]) * ni_ref[...] + b_ref[...]


_fin = pl.pallas_call(
    _fin_body,
    out_shape=jax.ShapeDtypeStruct((N_PAD, D), jnp.float32),
)


def kernel(feat, edge_index, W1, b1, W2, b2):
    src = edge_index[0]
    dst = edge_index[1]
    pad = EPW_PAD - EPW
    s3 = jnp.pad(src.reshape(NW, EPW), ((0, 0), (0, pad)),
                 constant_values=N).reshape(NW, NCH_D, CH)
    d3 = jnp.pad(dst.reshape(NW, EPW), ((0, 0), (0, pad)),
                 constant_values=N).reshape(NW, NCH_D, CH)
    sd = jnp.stack((s3, d3), axis=2)
    ones_row = jnp.ones((CH,), jnp.float32)
    zrow = jnp.zeros((RPT,), jnp.float32)
    zrows = jnp.zeros((ECH, D), jnp.float32)
    featp = jnp.pad(feat, ((0, N_PAD - N), (0, 0)))

    do0, di0, do1, di1 = _deg_kernel(sd, ones_row, zrow)
    no, ni = _norm(do0, di0, do1, di1)
    no = no.reshape(N_PAD, 1)
    ni = ni.reshape(N_PAD, 1)

    h1 = _mm1(featp, W1, no)
    p1 = _agg_kernel(h1, sd, zrows)
    h2 = _mm2(p1, ni, b1.reshape(1, D), W2, no)
    p2 = _agg_kernel(h2, sd, zrows)
    outp = _fin(p2, ni, b2.reshape(1, D))
    return outp[:N]


# speedup vs baseline: 1.0025x; 1.0025x over previous
import functools

import jax
import jax.numpy as jnp
from jax import lax
from jax.experimental import pallas as pl
from jax.experimental.pallas import tpu as pltpu
from jax.experimental.pallas import tpu_sc as plsc

N = 10000
E = 320000
D = 128

NW = 32
NSUB = 16
EPW = E // NW
CH = 128
ECH = 128
NCHUNK = 80
EPW_PAD = NCHUNK * ECH
NCH_D = NCHUNK
N_PAD = 10112
RPT = N_PAD // NSUB

_mesh = plsc.VectorSubcoreMesh(core_axis_name="c", subcore_axis_name="s")


@functools.partial(
    pl.kernel,
    mesh=_mesh,
    out_type=tuple(jax.ShapeDtypeStruct((N_PAD,), jnp.float32)
                   for _ in range(4)),
    scratch_types=[
        pltpu.VMEM((NCH_D, CH), jnp.int32),
        pltpu.VMEM((NCH_D, CH), jnp.int32),
        pltpu.VMEM((CH,), jnp.float32),
        pltpu.VMEM((RPT,), jnp.float32),
        pltpu.VMEM_SHARED((N_PAD,), jnp.float32),
        pltpu.VMEM_SHARED((N_PAD,), jnp.float32),
    ],
)
def _deg_kernel(src_hbm, dst_hbm, ones_hbm, zrow_hbm,
                dego0_hbm, degi0_hbm, dego1_hbm, degi1_hbm,
                sidx, didx, ones_v, vbuf, dego, degi):
    c = lax.axis_index("c")
    s = lax.axis_index("s")
    wid = c * NSUB + s
    base = s * RPT
    pltpu.sync_copy(ones_hbm, ones_v)
    pltpu.sync_copy(zrow_hbm, vbuf)
    pltpu.sync_copy(vbuf, dego.at[pl.ds(base, RPT)])
    pltpu.sync_copy(vbuf, degi.at[pl.ds(base, RPT)])
    plsc.subcore_barrier()

    pltpu.sync_copy(src_hbm.at[wid], sidx)
    pltpu.sync_copy(dst_hbm.at[wid], didx)

    @pl.loop(0, NCH_D)
    def _(j):
        pltpu.sync_copy(ones_v, dego.at[sidx.at[j]], add=True)
        pltpu.sync_copy(ones_v, degi.at[didx.at[j]], add=True)

    plsc.subcore_barrier()

    @pl.when(c == 0)
    def _():
        pltpu.sync_copy(dego.at[pl.ds(base, RPT)], vbuf)
        pltpu.sync_copy(vbuf, dego0_hbm.at[pl.ds(base, RPT)])
        pltpu.sync_copy(degi.at[pl.ds(base, RPT)], vbuf)
        pltpu.sync_copy(vbuf, degi0_hbm.at[pl.ds(base, RPT)])

    @pl.when(c == 1)
    def _():
        pltpu.sync_copy(dego.at[pl.ds(base, RPT)], vbuf)
        pltpu.sync_copy(vbuf, dego1_hbm.at[pl.ds(base, RPT)])
        pltpu.sync_copy(degi.at[pl.ds(base, RPT)], vbuf)
        pltpu.sync_copy(vbuf, degi1_hbm.at[pl.ds(base, RPT)])


@functools.partial(
    pl.kernel,
    mesh=_mesh,
    out_type=jax.ShapeDtypeStruct((2, N_PAD, D), jnp.float32),
    scratch_types=[
        pltpu.VMEM((NCHUNK, ECH), jnp.int32),
        pltpu.VMEM((NCHUNK, ECH), jnp.int32),
        pltpu.VMEM((ECH, D), jnp.float32),
        pltpu.VMEM_SHARED((N_PAD, D), jnp.float32),
    ],
)
def _agg_kernel(h_hbm, src_hbm, dst_hbm, zrows_hbm, out_hbm,
                sidx, didx, rows, acc):
    c = lax.axis_index("c")
    s = lax.axis_index("s")
    wid = c * NSUB + s
    base = s * RPT
    pltpu.sync_copy(zrows_hbm, rows)
    for k in range(-(-RPT // ECH)):
        sz = min(ECH, RPT - k * ECH)
        pltpu.sync_copy(rows.at[pl.ds(0, sz)],
                        acc.at[pl.ds(base + k * ECH, sz)])
    plsc.subcore_barrier()

    pltpu.sync_copy(src_hbm.at[wid], sidx)
    pltpu.sync_copy(dst_hbm.at[wid], didx)

    @pl.loop(0, NCHUNK)
    def _(j):
        pltpu.sync_copy(h_hbm.at[sidx.at[j]], rows)
        pltpu.sync_copy(rows, acc.at[didx.at[j]], add=True)

    plsc.subcore_barrier()
    for k in range(-(-RPT // ECH)):
        sz = min(ECH, RPT - k * ECH)
        pltpu.sync_copy(acc.at[pl.ds(base + k * ECH, sz)],
                        rows.at[pl.ds(0, sz)])
        pltpu.sync_copy(rows.at[pl.ds(0, sz)],
                        out_hbm.at[c, pl.ds(base + k * ECH, sz)])


def _norm_body(do0, di0, do1, di1, no_ref, ni_ref):
    no_ref[...] = lax.rsqrt(jnp.maximum(do0[...] + do1[...], 1.0))
    ni_ref[...] = lax.rsqrt(jnp.maximum(di0[...] + di1[...], 1.0))


_norm = pl.pallas_call(
    _norm_body,
    out_shape=(jax.ShapeDtypeStruct((N_PAD,), jnp.float32),
               jax.ShapeDtypeStruct((N_PAD,), jnp.float32)),
)


def _mm1_body(x_ref, w_ref, no_ref, o_ref):
    h = jnp.dot(x_ref[...], w_ref[...], preferred_element_type=jnp.float32)
    o_ref[...] = h * no_ref[...]


_mm1 = pl.pallas_call(
    _mm1_body,
    out_shape=jax.ShapeDtypeStruct((N_PAD, D), jnp.float32),
)


def _mm2_body(p_ref, ni_ref, b_ref, w_ref, no_ref, o_ref):
    x = (p_ref[0] + p_ref[1]) * ni_ref[...] + b_ref[...]
    h = jnp.dot(x, w_ref[...], preferred_element_type=jnp.float32)
    o_ref[...] = h * no_ref[...]


_mm2 = pl.pallas_call(
    _mm2_body,
    out_shape=jax.ShapeDtypeStruct((N_PAD, D), jnp.float32),
)


def _fin_body(p_ref, ni_ref, b_ref, o_ref):
    o_ref[...] = (p_ref[0] + p_ref[1]) * ni_ref[...] + b_ref[...]


_fin = pl.pallas_call(
    _fin_body,
    out_shape=jax.ShapeDtypeStruct((N_PAD, D), jnp.float32),
)


def kernel(feat, edge_index, W1, b1, W2, b2):
    src = edge_index[0]
    dst = edge_index[1]
    pad = EPW_PAD - EPW
    s3 = jnp.pad(src.reshape(NW, EPW), ((0, 0), (0, pad)),
                 constant_values=N).reshape(NW, NCH_D, CH)
    d3 = jnp.pad(dst.reshape(NW, EPW), ((0, 0), (0, pad)),
                 constant_values=N).reshape(NW, NCH_D, CH)
    ones_row = jnp.ones((CH,), jnp.float32)
    zrow = jnp.zeros((RPT,), jnp.float32)
    zrows = jnp.zeros((ECH, D), jnp.float32)
    featp = jnp.pad(feat, ((0, N_PAD - N), (0, 0)))

    do0, di0, do1, di1 = _deg_kernel(s3, d3, ones_row, zrow)
    no, ni = _norm(do0, di0, do1, di1)
    no = no.reshape(N_PAD, 1)
    ni = ni.reshape(N_PAD, 1)

    h1 = _mm1(featp, W1, no)
    p1 = _agg_kernel(h1, s3, d3, zrows)
    h2 = _mm2(p1, ni, b1.reshape(1, D), W2, no)
    p2 = _agg_kernel(h2, s3, d3, zrows)
    outp = _fin(p2, ni, b2.reshape(1, D))
    return outp[:N]


# speedup vs baseline: 2.2430x; 2.2374x over previous
import functools

import jax
import jax.numpy as jnp
from jax import lax
from jax.experimental import pallas as pl
from jax.experimental.pallas import tpu as pltpu
from jax.experimental.pallas import tpu_sc as plsc

N = 10000
E = 320000
D = 128

NW = 32
NSUB = 16
EPW = E // NW
CH = 128
ECH = 128
NCHUNK = 80
EPW_PAD = NCHUNK * ECH
NCH_D = NCHUNK
N_PAD = 10112
RPT = N_PAD // NSUB

_mesh = plsc.VectorSubcoreMesh(core_axis_name="c", subcore_axis_name="s")


@functools.partial(
    pl.kernel,
    mesh=_mesh,
    out_type=tuple(jax.ShapeDtypeStruct((N_PAD,), jnp.float32)
                   for _ in range(4)),
    scratch_types=[
        pltpu.VMEM((NCH_D, CH), jnp.int32),
        pltpu.VMEM((NCH_D, CH), jnp.int32),
        pltpu.VMEM((CH,), jnp.float32),
        pltpu.VMEM((RPT,), jnp.float32),
        pltpu.VMEM_SHARED((N_PAD,), jnp.float32),
        pltpu.VMEM_SHARED((N_PAD,), jnp.float32),
    ],
)
def _deg_kernel(src_hbm, dst_hbm, ones_hbm, zrow_hbm,
                dego0_hbm, degi0_hbm, dego1_hbm, degi1_hbm,
                sidx, didx, ones_v, vbuf, dego, degi):
    c = lax.axis_index("c")
    s = lax.axis_index("s")
    wid = c * NSUB + s
    base = s * RPT
    pltpu.sync_copy(ones_hbm, ones_v)
    pltpu.sync_copy(zrow_hbm, vbuf)
    pltpu.sync_copy(vbuf, dego.at[pl.ds(base, RPT)])
    pltpu.sync_copy(vbuf, degi.at[pl.ds(base, RPT)])
    plsc.subcore_barrier()

    pltpu.sync_copy(src_hbm.at[wid], sidx)
    pltpu.sync_copy(dst_hbm.at[wid], didx)

    @pl.loop(0, NCH_D)
    def _(j):
        pltpu.sync_copy(ones_v, dego.at[sidx.at[j]], add=True)
        pltpu.sync_copy(ones_v, degi.at[didx.at[j]], add=True)

    plsc.subcore_barrier()

    @pl.when(c == 0)
    def _():
        pltpu.sync_copy(dego.at[pl.ds(base, RPT)], vbuf)
        pltpu.sync_copy(vbuf, dego0_hbm.at[pl.ds(base, RPT)])
        pltpu.sync_copy(degi.at[pl.ds(base, RPT)], vbuf)
        pltpu.sync_copy(vbuf, degi0_hbm.at[pl.ds(base, RPT)])

    @pl.when(c == 1)
    def _():
        pltpu.sync_copy(dego.at[pl.ds(base, RPT)], vbuf)
        pltpu.sync_copy(vbuf, dego1_hbm.at[pl.ds(base, RPT)])
        pltpu.sync_copy(degi.at[pl.ds(base, RPT)], vbuf)
        pltpu.sync_copy(vbuf, degi1_hbm.at[pl.ds(base, RPT)])


@functools.partial(
    pl.kernel,
    mesh=_mesh,
    out_type=jax.ShapeDtypeStruct((2, N_PAD, D), jnp.float32),
    scratch_types=[
        pltpu.VMEM((NCHUNK, ECH), jnp.int32),
        pltpu.VMEM((NCHUNK, ECH), jnp.int32),
        pltpu.VMEM((ECH, D), jnp.float32),
        pltpu.VMEM_SHARED((N_PAD, D), jnp.float32),
    ],
)
def _agg_kernel(h_hbm, src_hbm, dst_hbm, zrows_hbm, out_hbm,
                sidx, didx, rows, acc):
    c = lax.axis_index("c")
    s = lax.axis_index("s")
    wid = c * NSUB + s
    base = s * RPT
    pltpu.sync_copy(zrows_hbm, rows)
    for k in range(-(-RPT // ECH)):
        sz = min(ECH, RPT - k * ECH)
        pltpu.sync_copy(rows.at[pl.ds(0, sz)],
                        acc.at[pl.ds(base + k * ECH, sz)])
    plsc.subcore_barrier()

    pltpu.sync_copy(src_hbm.at[wid], sidx)
    pltpu.sync_copy(dst_hbm.at[wid], didx)

    @pl.loop(0, NCHUNK)
    def _(j):
        pltpu.sync_copy(h_hbm.at[sidx.at[j]], rows)
        pltpu.sync_copy(rows, acc.at[didx.at[j]], add=True)

    plsc.subcore_barrier()
    for k in range(-(-RPT // ECH)):
        sz = min(ECH, RPT - k * ECH)
        pltpu.sync_copy(acc.at[pl.ds(base + k * ECH, sz)],
                        rows.at[pl.ds(0, sz)])
        pltpu.sync_copy(rows.at[pl.ds(0, sz)],
                        out_hbm.at[c, pl.ds(base + k * ECH, sz)])


def _norm_body(do0, di0, do1, di1, no_ref, ni_ref):
    no_ref[...] = lax.rsqrt(jnp.maximum(do0[...] + do1[...], 1.0))
    ni_ref[...] = lax.rsqrt(jnp.maximum(di0[...] + di1[...], 1.0))


_norm = pl.pallas_call(
    _norm_body,
    out_shape=(jax.ShapeDtypeStruct((N_PAD,), jnp.float32),
               jax.ShapeDtypeStruct((N_PAD,), jnp.float32)),
)


def _mm1_body(x_ref, w_ref, no_ref, o_ref):
    h = jnp.dot(x_ref[...], w_ref[...], preferred_element_type=jnp.float32)
    o_ref[...] = h * no_ref[...]


_mm1 = pl.pallas_call(
    _mm1_body,
    out_shape=jax.ShapeDtypeStruct((N_PAD, D), jnp.float32),
)


def _mm2_body(p_ref, ni_ref, b_ref, w_ref, no_ref, o_ref):
    x = (p_ref[0] + p_ref[1]) * ni_ref[...] + b_ref[...]
    h = jnp.dot(x, w_ref[...], preferred_element_type=jnp.float32)
    o_ref[...] = h * no_ref[...]


_mm2 = pl.pallas_call(
    _mm2_body,
    out_shape=jax.ShapeDtypeStruct((N_PAD, D), jnp.float32),
)


def _fin_body(p_ref, ni_ref, b_ref, o_ref):
    o_ref[...] = (p_ref[0] + p_ref[1]) * ni_ref[...] + b_ref[...]


_fin = pl.pallas_call(
    _fin_body,
    out_shape=jax.ShapeDtypeStruct((N_PAD, D), jnp.float32),
)


def kernel(feat, edge_index, W1, b1, W2, b2):
    src = edge_index[0]
    dst = edge_index[1]
    pad = EPW_PAD - EPW
    padv = jnp.broadcast_to(
        N + (jnp.arange(pad, dtype=jnp.int32) % (N_PAD - N)), (NW, pad))
    s3 = jnp.concatenate([src.reshape(NW, EPW), padv],
                         axis=1).reshape(NW, NCH_D, CH)
    d3 = jnp.concatenate([dst.reshape(NW, EPW), padv],
                         axis=1).reshape(NW, NCH_D, CH)
    ones_row = jnp.ones((CH,), jnp.float32)
    zrow = jnp.zeros((RPT,), jnp.float32)
    zrows = jnp.zeros((ECH, D), jnp.float32)
    featp = jnp.pad(feat, ((0, N_PAD - N), (0, 0)))

    do0, di0, do1, di1 = _deg_kernel(s3, d3, ones_row, zrow)
    no, ni = _norm(do0, di0, do1, di1)
    no = no.reshape(N_PAD, 1)
    ni = ni.reshape(N_PAD, 1)

    h1 = _mm1(featp, W1, no)
    p1 = _agg_kernel(h1, s3, d3, zrows)
    h2 = _mm2(p1, ni, b1.reshape(1, D), W2, no)
    p2 = _agg_kernel(h2, s3, d3, zrows)
    outp = _fin(p2, ni, b2.reshape(1, D))
    return outp[:N]


# speedup vs baseline: 2.4271x; 1.0820x over previous
import functools

import jax
import jax.numpy as jnp
from jax import lax
from jax.experimental import pallas as pl
from jax.experimental.pallas import tpu as pltpu
from jax.experimental.pallas import tpu_sc as plsc

N = 10000
E = 320000
D = 128

NW = 32
NSUB = 16
EPW = E // NW
CH = 128
ECH = 128
NCHUNK = 80
EPW_PAD = NCHUNK * ECH
NCH_D = NCHUNK
GSC = 8
N_PAD = 10112
RPT = N_PAD // NSUB

_mesh = plsc.VectorSubcoreMesh(core_axis_name="c", subcore_axis_name="s")


@functools.partial(
    pl.kernel,
    mesh=_mesh,
    out_type=tuple(jax.ShapeDtypeStruct((N_PAD,), jnp.float32)
                   for _ in range(4)),
    scratch_types=[
        pltpu.VMEM((NCH_D, CH), jnp.int32),
        pltpu.VMEM((NCH_D, CH), jnp.int32),
        pltpu.VMEM((CH,), jnp.float32),
        pltpu.VMEM((RPT,), jnp.float32),
        pltpu.VMEM_SHARED((N_PAD,), jnp.float32),
        pltpu.VMEM_SHARED((N_PAD,), jnp.float32),
    ],
)
def _deg_kernel(src_hbm, dst_hbm, ones_hbm, zrow_hbm,
                dego0_hbm, degi0_hbm, dego1_hbm, degi1_hbm,
                sidx, didx, ones_v, vbuf, dego, degi):
    c = lax.axis_index("c")
    s = lax.axis_index("s")
    wid = c * NSUB + s
    base = s * RPT
    pltpu.sync_copy(ones_hbm, ones_v)
    pltpu.sync_copy(zrow_hbm, vbuf)
    pltpu.sync_copy(vbuf, dego.at[pl.ds(base, RPT)])
    pltpu.sync_copy(vbuf, degi.at[pl.ds(base, RPT)])
    plsc.subcore_barrier()

    pltpu.sync_copy(src_hbm.at[wid], sidx)
    pltpu.sync_copy(dst_hbm.at[wid], didx)

    @pl.loop(0, NCH_D)
    def _(j):
        pltpu.sync_copy(ones_v, dego.at[sidx.at[j]], add=True)
        pltpu.sync_copy(ones_v, degi.at[didx.at[j]], add=True)

    plsc.subcore_barrier()

    @pl.when(c == 0)
    def _():
        pltpu.sync_copy(dego.at[pl.ds(base, RPT)], vbuf)
        pltpu.sync_copy(vbuf, dego0_hbm.at[pl.ds(base, RPT)])
        pltpu.sync_copy(degi.at[pl.ds(base, RPT)], vbuf)
        pltpu.sync_copy(vbuf, degi0_hbm.at[pl.ds(base, RPT)])

    @pl.when(c == 1)
    def _():
        pltpu.sync_copy(dego.at[pl.ds(base, RPT)], vbuf)
        pltpu.sync_copy(vbuf, dego1_hbm.at[pl.ds(base, RPT)])
        pltpu.sync_copy(degi.at[pl.ds(base, RPT)], vbuf)
        pltpu.sync_copy(vbuf, degi1_hbm.at[pl.ds(base, RPT)])


@functools.partial(
    pl.kernel,
    mesh=_mesh,
    out_type=jax.ShapeDtypeStruct((2, N_PAD, D), jnp.float32),
    scratch_types=[
        pltpu.VMEM((GSC, ECH), jnp.int32),
        pltpu.VMEM((GSC, ECH), jnp.int32),
        pltpu.VMEM((2, ECH, D), jnp.float32),
        pltpu.VMEM_SHARED((N_PAD, D), jnp.float32),
        pltpu.SemaphoreType.DMA((2,)),
        pltpu.SemaphoreType.DMA((2,)),
    ],
)
def _agg_kernel(h_hbm, src_hbm, dst_hbm, zrows_hbm, out_hbm,
                sidx, didx, rows, acc, gsem, ssem):
    c = lax.axis_index("c")
    s = lax.axis_index("s")
    wid = c * NSUB + s
    base = s * RPT
    pltpu.sync_copy(zrows_hbm, rows.at[0])
    for k in range(-(-RPT // ECH)):
        sz = min(ECH, RPT - k * ECH)
        pltpu.sync_copy(rows.at[0, pl.ds(0, sz)],
                        acc.at[pl.ds(base + k * ECH, sz)])
    plsc.subcore_barrier()

    @pl.loop(0, NCHUNK, step=GSC)
    def _(j0):
        pltpu.sync_copy(src_hbm.at[wid, pl.ds(j0, GSC)], sidx)
        pltpu.sync_copy(dst_hbm.at[wid, pl.ds(j0, GSC)], didx)
        for v in range(GSC // 2):
            u0, u1 = 2 * v, 2 * v + 1
            g0 = pltpu.async_copy(h_hbm.at[sidx.at[u0]], rows.at[0],
                                  gsem.at[0])
            g1 = pltpu.async_copy(h_hbm.at[sidx.at[u1]], rows.at[1],
                                  gsem.at[1])
            g0.wait()
            s0 = pltpu.async_copy(rows.at[0], acc.at[didx.at[u0]],
                                  ssem.at[0], add=True)
            g1.wait()
            s1 = pltpu.async_copy(rows.at[1], acc.at[didx.at[u1]],
                                  ssem.at[1], add=True)
            s0.wait()
            s1.wait()

    plsc.subcore_barrier()
    for k in range(-(-RPT // ECH)):
        sz = min(ECH, RPT - k * ECH)
        pltpu.sync_copy(acc.at[pl.ds(base + k * ECH, sz)],
                        rows.at[0, pl.ds(0, sz)])
        pltpu.sync_copy(rows.at[0, pl.ds(0, sz)],
                        out_hbm.at[c, pl.ds(base + k * ECH, sz)])


def _norm_body(do0, di0, do1, di1, no_ref, ni_ref):
    no_ref[...] = lax.rsqrt(jnp.maximum(do0[...] + do1[...], 1.0))
    ni_ref[...] = lax.rsqrt(jnp.maximum(di0[...] + di1[...], 1.0))


_norm = pl.pallas_call(
    _norm_body,
    out_shape=(jax.ShapeDtypeStruct((N_PAD,), jnp.float32),
               jax.ShapeDtypeStruct((N_PAD,), jnp.float32)),
)


def _mm1_body(x_ref, w_ref, no_ref, o_ref):
    h = jnp.dot(x_ref[...], w_ref[...], preferred_element_type=jnp.float32)
    o_ref[...] = h * no_ref[...]


_mm1 = pl.pallas_call(
    _mm1_body,
    out_shape=jax.ShapeDtypeStruct((N_PAD, D), jnp.float32),
)


def _mm2_body(p_ref, ni_ref, b_ref, w_ref, no_ref, o_ref):
    x = (p_ref[0] + p_ref[1]) * ni_ref[...] + b_ref[...]
    h = jnp.dot(x, w_ref[...], preferred_element_type=jnp.float32)
    o_ref[...] = h * no_ref[...]


_mm2 = pl.pallas_call(
    _mm2_body,
    out_shape=jax.ShapeDtypeStruct((N_PAD, D), jnp.float32),
)


def _fin_body(p_ref, ni_ref, b_ref, o_ref):
    o_ref[...] = (p_ref[0] + p_ref[1]) * ni_ref[...] + b_ref[...]


_fin = pl.pallas_call(
    _fin_body,
    out_shape=jax.ShapeDtypeStruct((N_PAD, D), jnp.float32),
)


def kernel(feat, edge_index, W1, b1, W2, b2):
    src = edge_index[0]
    dst = edge_index[1]
    pad = EPW_PAD - EPW
    padv = jnp.broadcast_to(
        N + (jnp.arange(pad, dtype=jnp.int32) % (N_PAD - N)), (NW, pad))
    s3 = jnp.concatenate([src.reshape(NW, EPW), padv],
                         axis=1).reshape(NW, NCH_D, CH)
    d3 = jnp.concatenate([dst.reshape(NW, EPW), padv],
                         axis=1).reshape(NW, NCH_D, CH)
    ones_row = jnp.ones((CH,), jnp.float32)
    zrow = jnp.zeros((RPT,), jnp.float32)
    zrows = jnp.zeros((ECH, D), jnp.float32)
    featp = jnp.pad(feat, ((0, N_PAD - N), (0, 0)))

    do0, di0, do1, di1 = _deg_kernel(s3, d3, ones_row, zrow)
    no, ni = _norm(do0, di0, do1, di1)
    no = no.reshape(N_PAD, 1)
    ni = ni.reshape(N_PAD, 1)

    h1 = _mm1(featp, W1, no)
    p1 = _agg_kernel(h1, s3, d3, zrows)
    h2 = _mm2(p1, ni, b1.reshape(1, D), W2, no)
    p2 = _agg_kernel(h2, s3, d3, zrows)
    outp = _fin(p2, ni, b2.reshape(1, D))
    return outp[:N]


# speedup vs baseline: 2.5534x; 1.0521x over previous
import functools

import jax
import jax.numpy as jnp
from jax import lax
from jax.experimental import pallas as pl
from jax.experimental.pallas import tpu as pltpu
from jax.experimental.pallas import tpu_sc as plsc

N = 10000
E = 320000
D = 128

NW = 32
NSUB = 16
EPW = E // NW
CH = 128
ECH = 128
NCHUNK = 80
EPW_PAD = NCHUNK * ECH
NCH_D = NCHUNK
GSC = 8
N_PAD = 10112
RPT = N_PAD // NSUB

_mesh = plsc.VectorSubcoreMesh(core_axis_name="c", subcore_axis_name="s")


@functools.partial(
    pl.kernel,
    mesh=_mesh,
    out_type=tuple(jax.ShapeDtypeStruct((N_PAD,), jnp.float32)
                   for _ in range(4)),
    scratch_types=[
        pltpu.VMEM((NCH_D, CH), jnp.int32),
        pltpu.VMEM((NCH_D, CH), jnp.int32),
        pltpu.VMEM((CH,), jnp.float32),
        pltpu.VMEM((RPT,), jnp.float32),
        pltpu.VMEM_SHARED((N_PAD,), jnp.float32),
        pltpu.VMEM_SHARED((N_PAD,), jnp.float32),
        pltpu.SemaphoreType.DMA,
    ],
)
def _deg_kernel(src_hbm, dst_hbm, ones_hbm, zrow_hbm,
                dego0_hbm, degi0_hbm, dego1_hbm, degi1_hbm,
                sidx, didx, ones_v, vbuf, dego, degi, dsem):
    c = lax.axis_index("c")
    s = lax.axis_index("s")
    wid = c * NSUB + s
    base = s * RPT
    pltpu.sync_copy(ones_hbm, ones_v)
    pltpu.sync_copy(zrow_hbm, vbuf)
    pltpu.sync_copy(vbuf, dego.at[pl.ds(base, RPT)])
    pltpu.sync_copy(vbuf, degi.at[pl.ds(base, RPT)])
    plsc.subcore_barrier()

    pltpu.sync_copy(src_hbm.at[wid], sidx)
    pltpu.sync_copy(dst_hbm.at[wid], didx)

    @pl.loop(0, NCH_D, step=4)
    def _(j0):
        hs = []
        for u in range(4):
            hs.append(pltpu.async_copy(ones_v, dego.at[sidx.at[j0 + u]],
                                       dsem, add=True))
            hs.append(pltpu.async_copy(ones_v, degi.at[didx.at[j0 + u]],
                                       dsem, add=True))
        for h in hs:
            h.wait()

    plsc.subcore_barrier()

    @pl.when(c == 0)
    def _():
        pltpu.sync_copy(dego.at[pl.ds(base, RPT)], vbuf)
        pltpu.sync_copy(vbuf, dego0_hbm.at[pl.ds(base, RPT)])
        pltpu.sync_copy(degi.at[pl.ds(base, RPT)], vbuf)
        pltpu.sync_copy(vbuf, degi0_hbm.at[pl.ds(base, RPT)])

    @pl.when(c == 1)
    def _():
        pltpu.sync_copy(dego.at[pl.ds(base, RPT)], vbuf)
        pltpu.sync_copy(vbuf, dego1_hbm.at[pl.ds(base, RPT)])
        pltpu.sync_copy(degi.at[pl.ds(base, RPT)], vbuf)
        pltpu.sync_copy(vbuf, degi1_hbm.at[pl.ds(base, RPT)])


@functools.partial(
    pl.kernel,
    mesh=_mesh,
    out_type=jax.ShapeDtypeStruct((2, N_PAD, D), jnp.float32),
    scratch_types=[
        pltpu.VMEM((GSC, ECH), jnp.int32),
        pltpu.VMEM((GSC, ECH), jnp.int32),
        pltpu.VMEM((2, ECH, D), jnp.float32),
        pltpu.VMEM_SHARED((N_PAD, D), jnp.float32),
        pltpu.SemaphoreType.DMA((2,)),
        pltpu.SemaphoreType.DMA((2,)),
        pltpu.SemaphoreType.DMA((2,)),
    ],
)
def _agg_kernel(h_hbm, src_hbm, dst_hbm, zrows_hbm, out_hbm,
                sidx, didx, rows, acc, gsem, ssem, xsem):
    c = lax.axis_index("c")
    s = lax.axis_index("s")
    wid = c * NSUB + s
    base = s * RPT
    pltpu.sync_copy(zrows_hbm, rows.at[0])
    zs = []
    for k in range(-(-RPT // ECH)):
        sz = min(ECH, RPT - k * ECH)
        zs.append(pltpu.async_copy(rows.at[0, pl.ds(0, sz)],
                                   acc.at[pl.ds(base + k * ECH, sz)],
                                   xsem.at[0]))
    for z in zs:
        z.wait()
    plsc.subcore_barrier()

    @pl.loop(0, NCHUNK, step=GSC)
    def _(j0):
        i0 = pltpu.async_copy(src_hbm.at[wid, pl.ds(j0, GSC)], sidx,
                              xsem.at[0])
        i1 = pltpu.async_copy(dst_hbm.at[wid, pl.ds(j0, GSC)], didx,
                              xsem.at[1])
        i0.wait()
        i1.wait()
        for v in range(GSC // 2):
            u0, u1 = 2 * v, 2 * v + 1
            g0 = pltpu.async_copy(h_hbm.at[sidx.at[u0]], rows.at[0],
                                  gsem.at[0])
            g1 = pltpu.async_copy(h_hbm.at[sidx.at[u1]], rows.at[1],
                                  gsem.at[1])
            g0.wait()
            s0 = pltpu.async_copy(rows.at[0], acc.at[didx.at[u0]],
                                  ssem.at[0], add=True)
            g1.wait()
            s1 = pltpu.async_copy(rows.at[1], acc.at[didx.at[u1]],
                                  ssem.at[1], add=True)
            s0.wait()
            s1.wait()

    plsc.subcore_barrier()
    for k in range(-(-RPT // ECH)):
        sz = min(ECH, RPT - k * ECH)
        pltpu.sync_copy(acc.at[pl.ds(base + k * ECH, sz)],
                        rows.at[0, pl.ds(0, sz)])
        pltpu.sync_copy(rows.at[0, pl.ds(0, sz)],
                        out_hbm.at[c, pl.ds(base + k * ECH, sz)])


def _norm_body(do0, di0, do1, di1, no_ref, ni_ref):
    no_ref[...] = lax.rsqrt(jnp.maximum(do0[...] + do1[...], 1.0))
    ni_ref[...] = lax.rsqrt(jnp.maximum(di0[...] + di1[...], 1.0))


_norm = pl.pallas_call(
    _norm_body,
    out_shape=(jax.ShapeDtypeStruct((N_PAD,), jnp.float32),
               jax.ShapeDtypeStruct((N_PAD,), jnp.float32)),
)


def _mm1_body(x_ref, w_ref, no_ref, o_ref):
    h = jnp.dot(x_ref[...], w_ref[...], preferred_element_type=jnp.float32)
    o_ref[...] = h * no_ref[...]


_mm1 = pl.pallas_call(
    _mm1_body,
    out_shape=jax.ShapeDtypeStruct((N_PAD, D), jnp.float32),
)


def _mm2_body(p_ref, ni_ref, b_ref, w_ref, no_ref, o_ref):
    x = (p_ref[0] + p_ref[1]) * ni_ref[...] + b_ref[...]
    h = jnp.dot(x, w_ref[...], preferred_element_type=jnp.float32)
    o_ref[...] = h * no_ref[...]


_mm2 = pl.pallas_call(
    _mm2_body,
    out_shape=jax.ShapeDtypeStruct((N_PAD, D), jnp.float32),
)


def _fin_body(p_ref, ni_ref, b_ref, o_ref):
    o_ref[...] = (p_ref[0] + p_ref[1]) * ni_ref[...] + b_ref[...]


_fin = pl.pallas_call(
    _fin_body,
    out_shape=jax.ShapeDtypeStruct((N_PAD, D), jnp.float32),
)


def kernel(feat, edge_index, W1, b1, W2, b2):
    src = edge_index[0]
    dst = edge_index[1]
    pad = EPW_PAD - EPW
    padv = jnp.broadcast_to(
        N + (jnp.arange(pad, dtype=jnp.int32) % (N_PAD - N)), (NW, pad))
    s3 = jnp.concatenate([src.reshape(NW, EPW), padv],
                         axis=1).reshape(NW, NCH_D, CH)
    d3 = jnp.concatenate([dst.reshape(NW, EPW), padv],
                         axis=1).reshape(NW, NCH_D, CH)
    ones_row = jnp.ones((CH,), jnp.float32)
    zrow = jnp.zeros((RPT,), jnp.float32)
    zrows = jnp.zeros((ECH, D), jnp.float32)
    featp = jnp.pad(feat, ((0, N_PAD - N), (0, 0)))

    do0, di0, do1, di1 = _deg_kernel(s3, d3, ones_row, zrow)
    no, ni = _norm(do0, di0, do1, di1)
    no = no.reshape(N_PAD, 1)
    ni = ni.reshape(N_PAD, 1)

    h1 = _mm1(featp, W1, no)
    p1 = _agg_kernel(h1, s3, d3, zrows)
    h2 = _mm2(p1, ni, b1.reshape(1, D), W2, no)
    p2 = _agg_kernel(h2, s3, d3, zrows)
    outp = _fin(p2, ni, b2.reshape(1, D))
    return outp[:N]
